# Initial kernel scaffold; baseline (speedup 1.0000x reference)
import jax
import jax.numpy as jnp
from jax.experimental import pallas as pl


def kernel(x, edge_index, batch, params):
    raise NotImplementedError("write your pallas kernel here")


# speedup vs baseline: 10.0881x; 10.0881x over previous
import functools

import jax
import jax.numpy as jnp
from jax import lax
from jax.experimental import pallas as pl
from jax.experimental.pallas import tpu as pltpu
from jax.experimental.pallas import tpu_sc as plsc

N = 10000
E = 320000
G = 100
D = 128
NHEADS = 4
ALPHA = 0.5
R = 25
NCLS = 10
M = 160

NC, NS = 2, 16
NW = NC * NS
CH = 128
NPAD = 10240
DUMP = 10200
CNT_BASE = 10100
DUMP_CNT = 10230

_MESH = dict(core_axis_name="c", subcore_axis_name="s", num_cores=NC,
             num_subcores=NS)


def _ceil_div(a, b):
    return -(-a // b)


@functools.partial(jax.jit, static_argnames=("d", "k"))
def _sc_prop(table, src3, dst3, zeros, *, d, k):
    rows_pt = NPAD // NS
    mesh = plsc.VectorSubcoreMesh(**_MESH)

    @functools.partial(
        pl.kernel,
        out_type=jax.ShapeDtypeStruct((NC, NPAD, d), jnp.float32),
        mesh=mesh,
        scratch_types=[
            pltpu.VMEM((k, CH), jnp.int32),
            pltpu.VMEM((k, CH), jnp.int32),
            pltpu.VMEM((CH, d), jnp.float32),
            pltpu.VMEM_SHARED((NPAD, d), jnp.float32),
            pltpu.SemaphoreType.DMA,
        ],
    )
    def kfn(table_hbm, src_hbm, dst_hbm, zero_hbm, out_hbm,
            sidx, didx, buf, accum, sem):
        c = lax.axis_index("c")
        s = lax.axis_index("s")
        w = c * NS + s
        pltpu.sync_copy(zero_hbm, accum.at[pl.ds(s * rows_pt, rows_pt)])
        pltpu.sync_copy(src_hbm.at[w], sidx)
        pltpu.sync_copy(dst_hbm.at[w], didx)
        plsc.subcore_barrier()

        def body(j, carry):
            pltpu.async_copy(table_hbm.at[sidx.at[j]], buf, sem).wait()
            pltpu.sync_copy(buf, accum.at[didx.at[j]], add=True)
            return carry

        lax.fori_loop(0, k, body, 0)
        plsc.subcore_barrier()
        pltpu.sync_copy(accum.at[pl.ds(s * rows_pt, rows_pt)],
                        out_hbm.at[c, pl.ds(s * rows_pt, rows_pt)])

    return kfn(table, src3, dst3, zeros)


@functools.partial(jax.jit, static_argnames=("k",))
def _sc_counts(dst3, ones, zeros, *, k):
    rows_pt = NPAD // NS
    mesh = plsc.VectorSubcoreMesh(**_MESH)

    @functools.partial(
        pl.kernel,
        out_type=jax.ShapeDtypeStruct((NC, NPAD, 16), jnp.float32),
        mesh=mesh,
        scratch_types=[
            pltpu.VMEM((k, CH), jnp.int32),
            pltpu.VMEM((CH, 16), jnp.float32),
            pltpu.VMEM_SHARED((NPAD, 16), jnp.float32),
        ],
        compiler_params=pltpu.CompilerParams(use_tc_tiling_on_sc=False),
    )
    def kfn(dst_hbm, ones_hbm, zero_hbm, out_hbm, didx, buf, accum):
        c = lax.axis_index("c")
        s = lax.axis_index("s")
        w = c * NS + s
        pltpu.sync_copy(zero_hbm, accum.at[pl.ds(s * rows_pt, rows_pt)])
        pltpu.sync_copy(dst_hbm.at[w], didx)
        pltpu.sync_copy(ones_hbm, buf)
        plsc.subcore_barrier()

        def body(j, carry):
            pltpu.sync_copy(buf, accum.at[didx.at[j]], add=True)
            return carry

        lax.fori_loop(0, k, body, 0)
        plsc.subcore_barrier()
        pltpu.sync_copy(accum.at[pl.ds(s * rows_pt, rows_pt)],
                        out_hbm.at[c, pl.ds(s * rows_pt, rows_pt)])

    return kfn(dst3, ones, zeros)


@functools.partial(jax.jit, static_argnames=("k",))
def _sc_prop_narrow(t, src3, dst3, zeros, *, k):
    rows_pt = NPAD // NS
    mesh = plsc.VectorSubcoreMesh(**_MESH)

    @functools.partial(
        pl.kernel,
        out_type=jax.ShapeDtypeStruct((NC, NPAD, 16), jnp.float32),
        mesh=mesh,
        scratch_types=[
            pltpu.VMEM((N,), jnp.float32),
            pltpu.VMEM((k, CH), jnp.int32),
            pltpu.VMEM((k, CH), jnp.int32),
            pltpu.VMEM((CH, 16), jnp.float32),
            pltpu.VMEM_SHARED((NPAD, 16), jnp.float32),
        ],
        compiler_params=pltpu.CompilerParams(
            needs_layout_passes=False, use_tc_tiling_on_sc=False),
    )
    def kfn(t_hbm, src_hbm, dst_hbm, zero_hbm, out_hbm,
            tv, sidx, didx, buf, accum):
        c = lax.axis_index("c")
        s = lax.axis_index("s")
        w = c * NS + s
        pltpu.sync_copy(zero_hbm, accum.at[pl.ds(s * rows_pt, rows_pt)])
        pltpu.sync_copy(t_hbm, tv)
        pltpu.sync_copy(src_hbm.at[w], sidx)
        pltpu.sync_copy(dst_hbm.at[w], didx)
        pltpu.sync_copy(zero_hbm.at[pl.ds(0, CH)], buf)
        plsc.subcore_barrier()
        lane = lax.iota(jnp.int32, 16)
        col0 = jnp.zeros((16,), jnp.int32)

        def chunk(j, carry):
            def grp(g, c2):
                sv = sidx[j, pl.ds(g * 16, 16)]
                vals = plsc.load_gather(tv, [sv])
                plsc.store_scatter(buf, [g * 16 + lane, col0], vals)
                return c2

            lax.fori_loop(0, 8, grp, 0)
            pltpu.sync_copy(buf, accum.at[didx.at[j]], add=True)
            return carry

        lax.fori_loop(0, k, chunk, 0)
        plsc.subcore_barrier()
        pltpu.sync_copy(accum.at[pl.ds(s * rows_pt, rows_pt)],
                        out_hbm.at[c, pl.ds(s * rows_pt, rows_pt)])

    return kfn(t, src3, dst3, zeros)


@functools.partial(jax.jit, static_argnames=("k",))
def _sc_gather_narrow(t, idx3, *, k):
    mesh = plsc.VectorSubcoreMesh(**_MESH)

    @functools.partial(
        pl.kernel,
        out_type=jax.ShapeDtypeStruct((NW * k * CH,), jnp.float32),
        mesh=mesh,
        scratch_types=[
            pltpu.VMEM((N,), jnp.float32),
            pltpu.VMEM((k, CH), jnp.int32),
            pltpu.VMEM((k * CH,), jnp.float32),
        ],
        compiler_params=pltpu.CompilerParams(needs_layout_passes=False),
    )
    def kfn(t_hbm, idx_hbm, out_hbm, tv, idxb, obuf):
        c = lax.axis_index("c")
        s = lax.axis_index("s")
        w = c * NS + s
        pltpu.sync_copy(t_hbm, tv)
        pltpu.sync_copy(idx_hbm.at[w], idxb)

        def chunk(j, carry):
            def grp(g, c2):
                sv = idxb[j, pl.ds(g * 16, 16)]
                obuf[pl.ds(j * CH + g * 16, 16)] = plsc.load_gather(tv, [sv])
                return c2

            lax.fori_loop(0, 8, grp, 0)
            return carry

        lax.fori_loop(0, k, chunk, 0)
        pltpu.sync_copy(obuf, out_hbm.at[pl.ds(w * k * CH, k * CH)])

    return kfn(t, idx3)


@functools.partial(jax.jit, static_argnames=("d", "k"))
def _sc_gather(table, idx3, *, d, k):
    mesh = plsc.VectorSubcoreMesh(**_MESH)

    @functools.partial(
        pl.kernel,
        out_type=jax.ShapeDtypeStruct((NW * k * CH, d), jnp.float32),
        mesh=mesh,
        scratch_types=[
            pltpu.VMEM((k, CH), jnp.int32),
            pltpu.VMEM((CH, d), jnp.float32),
            pltpu.SemaphoreType.DMA,
        ],
    )
    def kfn(table_hbm, idx_hbm, out_hbm, idxb, buf, sem):
        c = lax.axis_index("c")
        s = lax.axis_index("s")
        w = c * NS + s
        pltpu.sync_copy(idx_hbm.at[w], idxb)

        def body(j, carry):
            pltpu.async_copy(table_hbm.at[idxb.at[j]], buf, sem).wait()
            pltpu.sync_copy(buf, out_hbm.at[pl.ds(w * k * CH + j * CH, CH)])
            return carry

        lax.fori_loop(0, k, body, 0)

    return kfn(table, idx3)


def _pad_to_tiles(v, fill, k):
    tot = NW * k * CH
    v = jnp.concatenate(
        [v.astype(jnp.int32),
         jnp.full((tot - v.shape[0],), fill, jnp.int32)])
    return v.reshape(NW, k, CH)


def kernel(x, edge_index, batch, params):
    p = params
    src, dst = edge_index[0], edge_index[1]

    k_e = _ceil_div(E, NW * CH)
    src3 = _pad_to_tiles(src, 0, k_e)
    dst3 = _pad_to_tiles(dst, DUMP, k_e)

    k_c = _ceil_div(E + N, NW * CH)
    cnt_dst = jnp.concatenate(
        [dst.astype(jnp.int32), batch.astype(jnp.int32) + CNT_BASE])
    cnt3 = _pad_to_tiles(cnt_dst, DUMP_CNT, k_c)
    ones16 = jnp.ones((CH, 16), jnp.float32)
    zeros16 = jnp.zeros((NPAD // NS, 16), jnp.float32)
    cnt_out = _sc_counts(cnt3, ones16, zeros16, k=k_c)
    cnt_sum = cnt_out[0] + cnt_out[1]
    deg = cnt_sum[:N, 0] + 1.0
    counts = cnt_sum[CNT_BASE:CNT_BASE + G, 0]
    dinv = lax.rsqrt(jnp.maximum(deg, 1e-12))

    zeros128 = jnp.zeros((NPAD // NS, D), jnp.float32)

    def conv(hw, b):
        hws = dinv[:, None] * hw
        zr = _sc_prop(hws, src3, dst3, zeros128, d=D, k=k_e)
        return dinv[:, None] * (zr[0, :N] + zr[1, :N] + hws) + b

    def conv_narrow(t1):
        ts = dinv * t1
        zr = _sc_prop_narrow(ts, src3, dst3, zeros16, k=k_e)
        return dinv * (zr[0, :N, 0] + zr[1, :N, 0] + ts)

    h = x @ p["W_enc"] + p["b_enc"]
    x1 = (jax.nn.relu(conv(h @ p["W_g1_0"], p["b_g1_0"]))
          + jax.nn.relu(conv(h @ p["W_g1_1"], p["b_g1_1"])))
    x2 = (jax.nn.relu(conv(x1 @ p["W_g2_0"], p["b_g2_0"]))
          + jax.nn.relu(conv(x1 @ p["W_g2_1"], p["b_g2_1"])))
    x3 = (jax.nn.relu(conv(x2 @ p["W_g3_0"], p["b_g3_0"]))
          + jax.nn.relu(conv(x2 @ p["W_g3_1"], p["b_g3_1"])))

    def score(xi):
        return (ALPHA * (xi @ p["W_ws1"] + p["b_ws1"])
                + (1 - ALPHA) * (conv_narrow((xi @ p["W_ws2"])[:, 0])[:, None]
                                 + p["b_ws2"]))

    wcat = jnp.concatenate([score(x1), score(x2), score(x3)], axis=-1)
    wsm = jax.nn.softmax(wcat, axis=-1)
    xm = wsm[:, 0:1] * x1 + wsm[:, 1:2] * x2 + wsm[:, 2:3] * x3

    K = conv(xm @ p["W_k"], p["b_k"])
    V = conv(xm @ p["W_v"], p["b_v"])
    s_score = (ALPHA * (xm @ p["W_ps1"] + p["b_ps1"])[:, 0]
               + (1 - ALPHA) * (conv_narrow((xm @ p["W_ps2"])[:, 0])
                                + p["b_ps2"][0]))

    starts = jnp.concatenate(
        [jnp.zeros((1,), jnp.float32), jnp.cumsum(counts)[:-1]])
    starts_i = starts.astype(jnp.int32)
    miota = jnp.arange(M, dtype=jnp.int32)[None, :]
    didx_d = jnp.clip(starts_i[:, None] + miota, 0, N - 1)
    mask = miota < counts.astype(jnp.int32)[:, None]

    k_g = _ceil_div(G * M, NW * CH)
    gidx3 = _pad_to_tiles(didx_d.reshape(-1), 0, k_g)
    kv = jnp.concatenate([K, V], axis=1)
    kvd = _sc_gather(kv, gidx3, d=2 * D, k=k_g)[:G * M].reshape(G, M, 2 * D)
    sd = _sc_gather_narrow(s_score, gidx3, k=k_g)[:G * M].reshape(G, M)

    dense_s = jnp.where(mask, sd, -1e30)
    dense_i = jnp.where(mask, didx_d, 0)
    vals, argp = lax.top_k(dense_s, R)
    sel = jnp.take_along_axis(dense_i, argp, axis=1)
    ok = vals > -1e29

    k_x = _ceil_div(G * R, NW * CH)
    sidx3 = _pad_to_tiles(sel.reshape(-1), 0, k_x)
    xrows = _sc_gather(xm, sidx3, d=D, k=k_x)[:G * R].reshape(G, R, D)
    xp = jnp.where(ok[:, :, None], xrows * jnp.tanh(vals)[:, :, None], 0.0)

    Q = xp @ p["W_q"] + p["b_q"]
    maskf = mask[:, :, None].astype(jnp.float32)
    Kd = kvd[:, :, :D] * maskf
    Vd = kvd[:, :, D:] * maskf
    dh = D // NHEADS
    Qh = Q.reshape(G, R, NHEADS, dh).transpose(0, 2, 1, 3)
    Kh = Kd.reshape(G, M, NHEADS, dh).transpose(0, 2, 1, 3)
    Vh = Vd.reshape(G, M, NHEADS, dh).transpose(0, 2, 1, 3)
    logits = jnp.einsum("ghrd,ghmd->ghrm", Qh, Kh) / jnp.sqrt(jnp.float32(D))
    logits = jnp.where(mask[:, None, None, :], logits, -1e30)
    A = jax.nn.softmax(logits, axis=-1)
    O = Qh + jnp.einsum("ghrm,ghmd->ghrd", A, Vh)
    O = O.transpose(0, 2, 1, 3).reshape(G, R, D)
    O2 = O + jax.nn.relu(O @ p["W_o"] + p["b_o"])

    gv = jnp.einsum("r,grd->gd", p["w_read"], O2) + p["b_read"]
    h1 = jax.nn.relu(gv @ p["W_l1"] + p["b_l1"])
    logp = jax.nn.log_softmax(h1 @ p["W_l2"] + p["b_l2"], axis=-1)
    return logp, gv
